# Initial kernel scaffold; baseline (speedup 1.0000x reference)
import jax
import jax.numpy as jnp
from jax.experimental import pallas as pl


def kernel(logits, positions, aa_idx):
    raise NotImplementedError("write your pallas kernel here")


# speedup vs baseline: 85.7650x; 85.7650x over previous
import jax
import jax.numpy as jnp
from jax import lax
from jax.experimental import pallas as pl
from jax.experimental.pallas import tpu as pltpu
from jax.experimental.pallas import tpu_sc as plsc

LENGTH = 4096
NUM_AA = 20
BATCH = 16384
N_MUT = 32
TABLE = LENGTH * NUM_AA

NC, NS, LANES = 2, 16, 16
NW = NC * NS
B_PER_W = BATCH // NW
G_PER_W = B_PER_W // LANES


def _tc_prep(logits_ref, pos_ref, aa_ref, table_ref, fidx_t_ref):
    x = logits_ref[...]
    x = x - jnp.max(x, axis=-1, keepdims=True)
    lse = jnp.log(jnp.sum(jnp.exp(x), axis=-1, keepdims=True))
    table_ref[...] = x - lse
    fidx = pos_ref[...] * NUM_AA + aa_ref[...]
    fidx_t_ref[...] = fidx.T


def _sc_body(table_hbm, fidx_hbm, out_hbm, table_v, fidx_v, out_v):
    wid = lax.axis_index("s") * NC + lax.axis_index("c")
    base = wid * B_PER_W
    pltpu.sync_copy(table_hbm, table_v)
    pltpu.sync_copy(fidx_hbm.at[:, pl.ds(base, B_PER_W)], fidx_v)

    def group(g, carry):
        acc = jnp.zeros((LANES,), jnp.float32)
        for m in range(N_MUT):
            idx = fidx_v[m, pl.ds(g * LANES, LANES)]
            acc = acc + plsc.load_gather(table_v, [idx])
        out_v[pl.ds(g * LANES, LANES)] = acc * (1.0 / N_MUT)
        return carry

    lax.fori_loop(0, G_PER_W, group, 0)
    pltpu.sync_copy(out_v, out_hbm.at[pl.ds(base, B_PER_W)])


import functools


@functools.cache
def _sc_call():
    return pl.kernel(
        _sc_body,
        out_type=jax.ShapeDtypeStruct((BATCH,), jnp.float32),
        mesh=plsc.VectorSubcoreMesh(
            core_axis_name="c", subcore_axis_name="s",
            num_cores=NC, num_subcores=NS,
        ),
        scratch_types=[
            pltpu.VMEM((TABLE,), jnp.float32),
            pltpu.VMEM((N_MUT, B_PER_W), jnp.int32),
            pltpu.VMEM((B_PER_W,), jnp.float32),
        ],
        compiler_params=pltpu.CompilerParams(needs_layout_passes=False),
    )


def kernel(logits, positions, aa_idx):
    table, fidx_t = pl.pallas_call(
        _tc_prep,
        out_shape=(
            jax.ShapeDtypeStruct((LENGTH, NUM_AA), jnp.float32),
            jax.ShapeDtypeStruct((N_MUT, BATCH), jnp.int32),
        ),
    )(logits, positions, aa_idx)
    return _sc_call()(table.reshape(TABLE), fidx_t)
